# baseline (speedup 1.0000x reference)
import functools

import numpy as np
import jax
import jax.numpy as jnp
from jax import lax
from jax.experimental import pallas as pl
from jax.experimental.pallas import tpu as pltpu
from jax.experimental.pallas import tpu_sc as plsc

N = 128
B, D = 8, 512
R = B * D
NC, NS = 2, 16
NW = NC * NS
ROWS_PER_W = R // NW
POOLING_COUNTS = [15, 8, 8, 8]

LANES = 16
PAD = 64
ZSLOT = 0
CSLEN = PAD + N


def _build_tables():
    sidx, eidx, w, dest = [], [], [], []

    def add(starts, ends, ii, jj):
        for s, e, i, j in zip(starts, ends, ii, jj):
            sidx.append(PAD + s - 1 if s > 0 else ZSLOT)
            eidx.append(PAD + e - 1)
            w.append(1.0 / float(e - s))
            dest.append(i * N + j)

    diag = np.arange(N)
    add(diag, diag + 1, diag, diag)

    stride, offset = 1, 0
    for c in POOLING_COUNTS:
        for _ in range(c):
            offset += stride
            i = np.arange(0, N - offset, stride)
            j = np.arange(offset, N, stride)
            m = len(i)
            ks = np.arange(m)
            starts = (ks * N) // m
            ends = ((ks + 1) * N + m - 1) // m
            add(starts, ends, i, j)
        stride *= 2

    n = len(dest)
    npad = (-n) % LANES
    for k in range(npad):
        sidx.append(ZSLOT)
        eidx.append(ZSLOT)
        w.append(0.0)
        dest.append(N * N + k)
    return (np.asarray(sidx, np.int32), np.asarray(eidx, np.int32),
            np.asarray(w, np.float32), np.asarray(dest, np.int32))


_SIDX, _EIDX, _W, _DEST = _build_tables()
NT = _SIDX.shape[0]
NCHUNK = NT // LANES
TILE = N * N + LANES


NBUF = 2
RPB = 2
BTILE = RPB * N * N + LANES


def _body(x_hbm, s_hbm, e_hbm, w_hbm, d_hbm, out_hbm,
          x_v, csa_v, csb_v, t0_v, t1_v,
          s_v, e_v, w_v, d_v, sem0, sem1):
    wid = lax.axis_index("c") * NS + lax.axis_index("s")
    base = wid * ROWS_PER_W
    tiles = (t0_v, t1_v)
    sems = (sem0, sem1)

    pltpu.sync_copy(x_hbm.at[pl.ds(base * N, ROWS_PER_W * N)], x_v)
    pltpu.sync_copy(s_hbm, s_v)
    pltpu.sync_copy(e_hbm, e_v)
    pltpu.sync_copy(w_hbm, w_v)
    pltpu.sync_copy(d_hbm, d_v)

    zeros = jnp.zeros((LANES,), jnp.float32)
    for c in range(PAD // LANES):
        csa_v[pl.ds(c * LANES, LANES)] = zeros
        csb_v[pl.ds(c * LANES, LANES)] = zeros

    def zero_body(i, _):
        for t in tiles:
            t[pl.ds(i * LANES, LANES)] = zeros
        return _
    lax.fori_loop(0, BTILE // LANES, zero_body, None)

    def compute_row(row, tile, dof):
        for c in range(N // LANES):
            csa_v[pl.ds(PAD + c * LANES, LANES)] = (
                x_v[pl.ds(row * N + c * LANES, LANES)])

        src, dst = csa_v, csb_v
        k = 1
        while k < N:
            for c in range(N // LANES):
                off = PAD + c * LANES
                dst[pl.ds(off, LANES)] = (src[pl.ds(off, LANES)] +
                                          src[pl.ds(off - k, LANES)])
            src, dst = dst, src
            k *= 2
        cs = src

        @plsc.parallel_loop(0, NCHUNK, step=1, unroll=8)
        def _scat(t):
            off = t * LANES
            ei = e_v[pl.ds(off, LANES)]
            si = s_v[pl.ds(off, LANES)]
            wv = w_v[pl.ds(off, LANES)]
            dv = d_v[pl.ds(off, LANES)]
            if dof:
                dv = dv + dof
            a = plsc.load_gather(cs, [ei])
            b = plsc.load_gather(cs, [si])
            plsc.store_scatter(tile, [dv], (a - b) * wv, mask=wv > 0.0)

    def fill_buf(grp, b):
        for r in range(RPB):
            compute_row(grp * RPB + r, tiles[b], r * N * N)
        pltpu.async_copy(
            tiles[b].at[pl.ds(0, RPB * N * N)],
            out_hbm.at[pl.ds((base + grp * RPB) * (N * N), RPB * N * N)],
            sems[b])

    def wait_out_dma(b):
        pltpu.make_async_copy(
            tiles[b].at[pl.ds(0, RPB * N * N)],
            out_hbm.at[pl.ds(base * (N * N), RPB * N * N)], sems[b]).wait()

    for b in range(NBUF):
        fill_buf(jnp.int32(b), b)

    def it_body(it, _):
        for b in range(NBUF):
            wait_out_dma(b)
            fill_buf(it * NBUF + b, b)
        return _
    lax.fori_loop(1, ROWS_PER_W // (NBUF * RPB), it_body, None)

    for b in range(NBUF):
        wait_out_dma(b)


def kernel(x):
    X = x.reshape(R * N)
    mesh = plsc.VectorSubcoreMesh(core_axis_name="c", subcore_axis_name="s",
                                  num_cores=NC, num_subcores=NS)
    run = functools.partial(
        pl.kernel,
        out_type=jax.ShapeDtypeStruct((R * N * N,), jnp.float32),
        mesh=mesh,
        compiler_params=pltpu.CompilerParams(needs_layout_passes=False),
        scratch_types=[
            pltpu.VMEM((ROWS_PER_W * N,), jnp.float32),
            pltpu.VMEM((CSLEN,), jnp.float32),
            pltpu.VMEM((CSLEN,), jnp.float32),
            pltpu.VMEM((BTILE,), jnp.float32),
            pltpu.VMEM((BTILE,), jnp.float32),
            pltpu.VMEM((NT,), jnp.int32),
            pltpu.VMEM((NT,), jnp.int32),
            pltpu.VMEM((NT,), jnp.float32),
            pltpu.VMEM((NT,), jnp.int32),
            pltpu.SemaphoreType.DMA,
            pltpu.SemaphoreType.DMA,
        ],
    )(_body)
    out = run(X, jnp.asarray(_SIDX), jnp.asarray(_EIDX),
              jnp.asarray(_W), jnp.asarray(_DEST))
    return out.reshape(B, D, N, N)


# speedup vs baseline: 1.0385x; 1.0385x over previous
import functools

import numpy as np
import jax
import jax.numpy as jnp
from jax import lax
from jax.experimental import pallas as pl
from jax.experimental.pallas import tpu as pltpu
from jax.experimental.pallas import tpu_sc as plsc

N = 128
B, D = 8, 512
R = B * D
NC, NS = 2, 16
NW = NC * NS
ROWS_PER_W = R // NW
POOLING_COUNTS = [15, 8, 8, 8]

LANES = 16
PAD = 64
ZSLOT = 0
CSLEN = PAD + N


def _build_tables():
    sidx, eidx, w, dest = [], [], [], []

    def add(starts, ends, ii, jj):
        for s, e, i, j in zip(starts, ends, ii, jj):
            sidx.append(PAD + s - 1 if s > 0 else ZSLOT)
            eidx.append(PAD + e - 1)
            w.append(1.0 / float(e - s))
            dest.append(i * N + j)

    diag = np.arange(N)
    add(diag, diag + 1, diag, diag)

    stride, offset = 1, 0
    for c in POOLING_COUNTS:
        for _ in range(c):
            offset += stride
            i = np.arange(0, N - offset, stride)
            j = np.arange(offset, N, stride)
            m = len(i)
            ks = np.arange(m)
            starts = (ks * N) // m
            ends = ((ks + 1) * N + m - 1) // m
            add(starts, ends, i, j)
        stride *= 2

    n = len(dest)
    npad = (-n) % LANES
    for k in range(npad):
        sidx.append(ZSLOT)
        eidx.append(ZSLOT)
        w.append(0.0)
        dest.append(N * N + k)
    return (np.asarray(sidx, np.int32), np.asarray(eidx, np.int32),
            np.asarray(w, np.float32), np.asarray(dest, np.int32))


_SIDX, _EIDX, _W, _DEST = _build_tables()
NT = _SIDX.shape[0]
NCHUNK = NT // LANES
TILE = N * N + LANES


NBUF = 5


def _body(x_hbm, s_hbm, e_hbm, w_hbm, d_hbm, out_hbm,
          x_v, csa_v, csb_v, t0_v, t1_v, t2_v, t3_v, t4_v,
          s_v, e_v, w_v, d_v, sem0, sem1, sem2, sem3, sem4):
    wid = lax.axis_index("c") * NS + lax.axis_index("s")
    base = wid * ROWS_PER_W
    tiles = (t0_v, t1_v, t2_v, t3_v, t4_v)
    sems = (sem0, sem1, sem2, sem3, sem4)

    pltpu.sync_copy(x_hbm.at[pl.ds(base * N, ROWS_PER_W * N)], x_v)
    pltpu.sync_copy(s_hbm, s_v)
    pltpu.sync_copy(e_hbm, e_v)
    pltpu.sync_copy(w_hbm, w_v)
    pltpu.sync_copy(d_hbm, d_v)

    zeros = jnp.zeros((LANES,), jnp.float32)
    for c in range(PAD // LANES):
        csa_v[pl.ds(c * LANES, LANES)] = zeros
        csb_v[pl.ds(c * LANES, LANES)] = zeros

    def zero_body(i, _):
        for t in tiles:
            t[pl.ds(i * LANES, LANES)] = zeros
        return _
    lax.fori_loop(0, TILE // LANES, zero_body, None)

    def compute_row(row, tile):
        for c in range(N // LANES):
            csa_v[pl.ds(PAD + c * LANES, LANES)] = (
                x_v[pl.ds(row * N + c * LANES, LANES)])

        src, dst = csa_v, csb_v
        k = 1
        while k < N:
            for c in range(N // LANES):
                off = PAD + c * LANES
                dst[pl.ds(off, LANES)] = (src[pl.ds(off, LANES)] +
                                          src[pl.ds(off - k, LANES)])
            src, dst = dst, src
            k *= 2
        cs = src

        @plsc.parallel_loop(0, NCHUNK, step=1, unroll=8)
        def _scat(t):
            off = t * LANES
            ei = e_v[pl.ds(off, LANES)]
            si = s_v[pl.ds(off, LANES)]
            wv = w_v[pl.ds(off, LANES)]
            dv = d_v[pl.ds(off, LANES)]
            a = plsc.load_gather(cs, [ei])
            b = plsc.load_gather(cs, [si])
            plsc.store_scatter(tile, [dv], (a - b) * wv)

    def start_out_dma(row, b):
        pltpu.async_copy(
            tiles[b].at[pl.ds(0, N * N)],
            out_hbm.at[pl.ds((base + row) * (N * N), N * N)], sems[b])

    def wait_out_dma(b):
        pltpu.make_async_copy(
            tiles[b].at[pl.ds(0, N * N)],
            out_hbm.at[pl.ds(base * (N * N), N * N)], sems[b]).wait()

    for b in range(NBUF):
        compute_row(jnp.int32(b), tiles[b])
        start_out_dma(jnp.int32(b), b)

    def it_body(it, _):
        for b in range(NBUF):
            row = it * NBUF + b
            wait_out_dma(b)
            compute_row(row, tiles[b])
            start_out_dma(row, b)
        return _
    lax.fori_loop(1, ROWS_PER_W // NBUF, it_body, None)

    tail_base = (ROWS_PER_W // NBUF) * NBUF
    for b in range(ROWS_PER_W - tail_base):
        wait_out_dma(b)
        compute_row(jnp.int32(tail_base + b), tiles[b])
        start_out_dma(jnp.int32(tail_base + b), b)

    for b in range(NBUF):
        wait_out_dma(b)


def kernel(x):
    X = x.reshape(R * N)
    mesh = plsc.VectorSubcoreMesh(core_axis_name="c", subcore_axis_name="s",
                                  num_cores=NC, num_subcores=NS)
    run = functools.partial(
        pl.kernel,
        out_type=jax.ShapeDtypeStruct((R * N * N,), jnp.float32),
        mesh=mesh,
        compiler_params=pltpu.CompilerParams(needs_layout_passes=False),
        scratch_types=[
            pltpu.VMEM((ROWS_PER_W * N,), jnp.float32),
            pltpu.VMEM((CSLEN,), jnp.float32),
            pltpu.VMEM((CSLEN,), jnp.float32),
            pltpu.VMEM((TILE,), jnp.float32),
            pltpu.VMEM((TILE,), jnp.float32),
            pltpu.VMEM((TILE,), jnp.float32),
            pltpu.VMEM((TILE,), jnp.float32),
            pltpu.VMEM((TILE,), jnp.float32),
            pltpu.VMEM((NT,), jnp.int32),
            pltpu.VMEM((NT,), jnp.int32),
            pltpu.VMEM((NT,), jnp.float32),
            pltpu.VMEM((NT,), jnp.int32),
            pltpu.SemaphoreType.DMA,
            pltpu.SemaphoreType.DMA,
            pltpu.SemaphoreType.DMA,
            pltpu.SemaphoreType.DMA,
            pltpu.SemaphoreType.DMA,
        ],
    )(_body)
    out = run(X, jnp.asarray(_SIDX), jnp.asarray(_EIDX),
              jnp.asarray(_W), jnp.asarray(_DEST))
    return out.reshape(B, D, N, N)


# speedup vs baseline: 1.0619x; 1.0226x over previous
import functools

import numpy as np
import jax
import jax.numpy as jnp
from jax import lax
from jax.experimental import pallas as pl
from jax.experimental.pallas import tpu as pltpu
from jax.experimental.pallas import tpu_sc as plsc

N = 128
B, D = 8, 512
R = B * D
NC, NS = 2, 16
NW = NC * NS
ROWS_PER_W = R // NW
POOLING_COUNTS = [15, 8, 8, 8]

LANES = 16
PAD = 64
ZSLOT = 0
CSLEN = PAD + N


def _build_tables():
    sidx, eidx, w, dest = [], [], [], []

    def add(starts, ends, ii, jj):
        for s, e, i, j in zip(starts, ends, ii, jj):
            sidx.append(PAD + s - 1 if s > 0 else ZSLOT)
            eidx.append(PAD + e - 1)
            w.append(1.0 / float(e - s))
            dest.append(i * N + j)

    diag = np.arange(N)
    add(diag, diag + 1, diag, diag)

    stride, offset = 1, 0
    for c in POOLING_COUNTS:
        for _ in range(c):
            offset += stride
            i = np.arange(0, N - offset, stride)
            j = np.arange(offset, N, stride)
            m = len(i)
            ks = np.arange(m)
            starts = (ks * N) // m
            ends = ((ks + 1) * N + m - 1) // m
            add(starts, ends, i, j)
        stride *= 2

    n = len(dest)
    npad = (-n) % LANES
    for k in range(npad):
        sidx.append(ZSLOT)
        eidx.append(ZSLOT)
        w.append(0.0)
        dest.append(N * N + k)
    return (np.asarray(sidx, np.int32), np.asarray(eidx, np.int32),
            np.asarray(w, np.float32), np.asarray(dest, np.int32))


_SIDX, _EIDX, _W, _DEST = _build_tables()
NT = _SIDX.shape[0]
NCHUNK = NT // LANES
TILE = N * N + LANES


NBUF = 4


def _body(x_hbm, s_hbm, e_hbm, w_hbm, d_hbm, out_hbm,
          x_v, csa_v, csb_v, t0_v, t1_v, t2_v, t3_v,
          s_v, e_v, w_v, d_v, sem0, sem1, sem2, sem3):
    wid = lax.axis_index("c") * NS + lax.axis_index("s")
    base = wid * ROWS_PER_W
    tiles = (t0_v, t1_v, t2_v, t3_v)
    sems = (sem0, sem1, sem2, sem3)

    pltpu.sync_copy(x_hbm.at[pl.ds(base * N, ROWS_PER_W * N)], x_v)
    pltpu.sync_copy(s_hbm, s_v)
    pltpu.sync_copy(e_hbm, e_v)
    pltpu.sync_copy(w_hbm, w_v)
    pltpu.sync_copy(d_hbm, d_v)

    zeros = jnp.zeros((LANES,), jnp.float32)
    for c in range(PAD // LANES):
        csa_v[pl.ds(c * LANES, LANES)] = zeros
        csb_v[pl.ds(c * LANES, LANES)] = zeros

    @plsc.parallel_loop(0, TILE // LANES, step=1, unroll=8)
    def _zero(i):
        for t in tiles:
            t[pl.ds(i * LANES, LANES)] = zeros

    def compute_row(row, tile):
        for c in range(N // LANES):
            csa_v[pl.ds(PAD + c * LANES, LANES)] = (
                x_v[pl.ds(row * N + c * LANES, LANES)])

        src, dst = csa_v, csb_v
        k = 1
        while k < N:
            for c in range(N // LANES):
                off = PAD + c * LANES
                dst[pl.ds(off, LANES)] = (src[pl.ds(off, LANES)] +
                                          src[pl.ds(off - k, LANES)])
            src, dst = dst, src
            k *= 2
        cs = src

        @plsc.parallel_loop(0, NCHUNK, step=1, unroll=8)
        def _scat(t):
            off = t * LANES
            ei = e_v[pl.ds(off, LANES)]
            si = s_v[pl.ds(off, LANES)]
            wv = w_v[pl.ds(off, LANES)]
            dv = d_v[pl.ds(off, LANES)]
            a = plsc.load_gather(cs, [ei])
            b = plsc.load_gather(cs, [si])
            plsc.store_scatter(tile, [dv], (a - b) * wv)

    def start_out_dma(row, b):
        pltpu.async_copy(
            tiles[b].at[pl.ds(0, N * N)],
            out_hbm.at[pl.ds((base + row) * (N * N), N * N)], sems[b])

    def wait_out_dma(b):
        pltpu.make_async_copy(
            tiles[b].at[pl.ds(0, N * N)],
            out_hbm.at[pl.ds(base * (N * N), N * N)], sems[b]).wait()

    for b in range(NBUF):
        compute_row(jnp.int32(b), tiles[b])
        start_out_dma(jnp.int32(b), b)

    def it_body(it, _):
        for b in range(NBUF):
            row = it * NBUF + b
            wait_out_dma(b)
            compute_row(row, tiles[b])
            start_out_dma(row, b)
        return _
    lax.fori_loop(1, ROWS_PER_W // NBUF, it_body, None)

    for b in range(NBUF):
        wait_out_dma(b)


def kernel(x):
    X = x.reshape(R * N)
    mesh = plsc.VectorSubcoreMesh(core_axis_name="c", subcore_axis_name="s",
                                  num_cores=NC, num_subcores=NS)
    run = functools.partial(
        pl.kernel,
        out_type=jax.ShapeDtypeStruct((R * N * N,), jnp.float32),
        mesh=mesh,
        compiler_params=pltpu.CompilerParams(needs_layout_passes=False),
        scratch_types=[
            pltpu.VMEM((ROWS_PER_W * N,), jnp.float32),
            pltpu.VMEM((CSLEN,), jnp.float32),
            pltpu.VMEM((CSLEN,), jnp.float32),
            pltpu.VMEM((TILE,), jnp.float32),
            pltpu.VMEM((TILE,), jnp.float32),
            pltpu.VMEM((TILE,), jnp.float32),
            pltpu.VMEM((TILE,), jnp.float32),
            pltpu.VMEM((NT,), jnp.int32),
            pltpu.VMEM((NT,), jnp.int32),
            pltpu.VMEM((NT,), jnp.float32),
            pltpu.VMEM((NT,), jnp.int32),
            pltpu.SemaphoreType.DMA,
            pltpu.SemaphoreType.DMA,
            pltpu.SemaphoreType.DMA,
            pltpu.SemaphoreType.DMA,
        ],
    )(_body)
    out = run(X, jnp.asarray(_SIDX), jnp.asarray(_EIDX),
              jnp.asarray(_W), jnp.asarray(_DEST))
    return out.reshape(B, D, N, N)
